# baseline (speedup 1.0000x reference)
import functools

import jax
import jax.numpy as jnp
from jax import lax
from jax.experimental import pallas as pl
from jax.experimental.pallas import tpu as pltpu
from jax.experimental.pallas import tpu_sc as plsc

N = 10000
E = 320000
IN_CH = 128
HIDDEN = 256
OUT_CH = 128

NC = 2
NS = 16
NW = NC * NS
NP = 10240
B = 32
S = 320
NB = 4
G = 2
EW = S * B
EPAD = NW * EW
RPT = NP // NS
CH = RPT // B

D1 = 144


def _make_segsum(D):
    mesh = plsc.VectorSubcoreMesh(core_axis_name="c", subcore_axis_name="s")

    @functools.partial(
        pl.kernel,
        mesh=mesh,
        out_type=jax.ShapeDtypeStruct((NC, NP, D), jnp.float32),
        scratch_types=[
            pltpu.VMEM((S, B), jnp.int32),
            pltpu.VMEM((S, B), jnp.int32),
            [pltpu.VMEM((B, D), jnp.float32) for _ in range(NB)],
            pltpu.VMEM_SHARED((NP, D), jnp.float32),
            [pltpu.SemaphoreType.DMA for _ in range(NB)],
            [pltpu.SemaphoreType.DMA for _ in range(NB)],
        ],
        compiler_params=pltpu.CompilerParams(use_tc_tiling_on_sc=False),
    )
    def seg(f_hbm, src_hbm, dst_hbm, zeros_hbm, out_hbm,
            src_v, dst_v, rows_v, acc_sh, gsems, ssems):
        c = lax.axis_index("c")
        s = lax.axis_index("s")
        w = c * NS + s

        pltpu.sync_copy(src_hbm.at[w], src_v)
        pltpu.sync_copy(dst_hbm.at[w], dst_v)

        pltpu.sync_copy(zeros_hbm.at[pl.ds(s * RPT, RPT)],
                        acc_sh.at[pl.ds(s * RPT, RPT)])
        plsc.subcore_barrier()

        for v in range(G):
            pltpu.async_copy(f_hbm.at[src_v.at[v]], rows_v[v], gsems[v])

        for b in range(NB):
            pltpu.make_async_copy(f_hbm.at[src_v.at[b]], rows_v[b],
                                  gsems[b]).wait()
            pltpu.async_copy(rows_v[b], acc_sh.at[dst_v.at[b]], ssems[b],
                             add=True)
            wst = b + G
            bw = wst % NB
            if wst >= NB:
                pltpu.make_async_copy(rows_v[bw],
                                      acc_sh.at[dst_v.at[wst - NB]],
                                      ssems[bw]).wait()
            pltpu.async_copy(f_hbm.at[src_v.at[wst]], rows_v[bw],
                             gsems[bw])

        def body(i, carry):
            for b in range(NB):
                v = NB * i + b
                pltpu.make_async_copy(f_hbm.at[src_v.at[v]], rows_v[b],
                                      gsems[b]).wait()
                pltpu.async_copy(rows_v[b], acc_sh.at[dst_v.at[v]],
                                 ssems[b], add=True)
                wst = v + G
                bw = (b + G) % NB

                @pl.when(wst < S)
                def _():
                    pltpu.make_async_copy(rows_v[bw],
                                          acc_sh.at[dst_v.at[wst - NB]],
                                          ssems[bw]).wait()
                    pltpu.async_copy(f_hbm.at[src_v.at[wst]], rows_v[bw],
                                     gsems[bw])
            return carry

        lax.fori_loop(1, S // NB, body, 0)

        for b in range(NB):
            pltpu.make_async_copy(rows_v[b],
                                  acc_sh.at[dst_v.at[S - NB + b]],
                                  ssems[b]).wait()
        plsc.subcore_barrier()

        pltpu.sync_copy(acc_sh.at[pl.ds(s * RPT, RPT)],
                        out_hbm.at[c, pl.ds(s * RPT, RPT)])

    return seg


_segsum_l1 = _make_segsum(D1)
_segsum_l2 = _make_segsum(OUT_CH)


_BN1 = 2048


def _tc1_body(p_ref, x_ref, wl1_ref, wr1_ref, b1_ref, wl2_ref, wr2_ref,
              b2_ref, y2_ref, r2_ref, inv_ref):
    acc = p_ref[0] + p_ref[1]
    inv = 1.0 / jnp.maximum(acc[:, IN_CH:IN_CH + 1], 1.0)
    agg = acc[:, :IN_CH] * inv
    h = jnp.maximum(
        jnp.dot(agg, wl1_ref[...], preferred_element_type=jnp.float32)
        + jnp.dot(x_ref[...], wr1_ref[...], preferred_element_type=jnp.float32)
        + b1_ref[...], 0.0)
    y2_ref[...] = jnp.dot(h, wl2_ref[...], preferred_element_type=jnp.float32)
    r2_ref[...] = (jnp.dot(h, wr2_ref[...], preferred_element_type=jnp.float32)
                   + b2_ref[...])
    inv_ref[...] = inv


def _tc1(p1, x, wl1, wr1, b1, wl2, wr2, b2):
    g = NP // _BN1
    return pl.pallas_call(
        _tc1_body,
        grid=(g,),
        in_specs=[
            pl.BlockSpec((NC, _BN1, D1), lambda i: (0, i, 0)),
            pl.BlockSpec((_BN1, IN_CH), lambda i: (i, 0)),
            pl.BlockSpec((IN_CH, HIDDEN), lambda i: (0, 0)),
            pl.BlockSpec((IN_CH, HIDDEN), lambda i: (0, 0)),
            pl.BlockSpec((1, HIDDEN), lambda i: (0, 0)),
            pl.BlockSpec((HIDDEN, OUT_CH), lambda i: (0, 0)),
            pl.BlockSpec((HIDDEN, OUT_CH), lambda i: (0, 0)),
            pl.BlockSpec((1, OUT_CH), lambda i: (0, 0)),
        ],
        out_specs=[
            pl.BlockSpec((_BN1, OUT_CH), lambda i: (i, 0)),
            pl.BlockSpec((_BN1, OUT_CH), lambda i: (i, 0)),
            pl.BlockSpec((_BN1, 1), lambda i: (i, 0)),
        ],
        out_shape=[
            jax.ShapeDtypeStruct((NP, OUT_CH), jnp.float32),
            jax.ShapeDtypeStruct((NP, OUT_CH), jnp.float32),
            jax.ShapeDtypeStruct((NP, 1), jnp.float32),
        ],
    )(p1, x, wl1, wr1, b1, wl2, wr2, b2)


_BN2 = 2000


def _tc2_body(p_ref, r2_ref, inv_ref, z_ref):
    z_ref[...] = (p_ref[0] + p_ref[1]) * inv_ref[...] + r2_ref[...]


def _tc2(p2, r2, inv):
    g = N // _BN2
    return pl.pallas_call(
        _tc2_body,
        grid=(g,),
        in_specs=[
            pl.BlockSpec((NC, _BN2, OUT_CH), lambda i: (0, i, 0)),
            pl.BlockSpec((_BN2, OUT_CH), lambda i: (i, 0)),
            pl.BlockSpec((_BN2, 1), lambda i: (i, 0)),
        ],
        out_specs=pl.BlockSpec((_BN2, OUT_CH), lambda i: (i, 0)),
        out_shape=jax.ShapeDtypeStruct((N, OUT_CH), jnp.float32),
    )(p2, r2, inv)


def kernel(x, edge_index, Wl1, Wr1, b1, Wl2, Wr2, b2):
    npad = EPAD - E
    fill = (N + (jnp.arange(npad, dtype=jnp.int32) % (NP - N)))
    src = jnp.concatenate([edge_index[0], fill]).reshape(NW, S, B)
    dst = jnp.concatenate([edge_index[1], fill]).reshape(NW, S, B)

    x_pad = jnp.pad(x, ((0, NP - N), (0, 0)))
    ones_col = jnp.pad(jnp.ones((N, 1), jnp.float32), ((0, NP - N), (0, 0)))
    x_aug = jnp.concatenate(
        [x_pad, ones_col, jnp.zeros((NP, D1 - IN_CH - 1), jnp.float32)],
        axis=1)
    zeros1 = jnp.zeros((NP, D1), jnp.float32)
    p1 = _segsum_l1(x_aug, src, dst, zeros1)

    y2, r2, inv = _tc1(p1, x_pad, Wl1, Wr1, b1.reshape(1, HIDDEN),
                       Wl2, Wr2, b2.reshape(1, OUT_CH))

    zeros2 = jnp.zeros((NP, OUT_CH), jnp.float32)
    p2 = _segsum_l2(y2, src, dst, zeros2)

    return _tc2(p2, r2, inv)


# speedup vs baseline: 1.3288x; 1.3288x over previous
import functools

import jax
import jax.numpy as jnp
from jax import lax
from jax.experimental import pallas as pl
from jax.experimental.pallas import tpu as pltpu
from jax.experimental.pallas import tpu_sc as plsc

N = 10000
E = 320000
IN_CH = 128
HIDDEN = 256
OUT_CH = 128

NC = 2
NS = 16
NW = NC * NS
NP = 10240
B = 40
S = 256
NBUF = 4
EW = S * B
EPAD = NW * EW
RPT = NP // NS
CH = RPT // B

D1 = 136


def _make_segsum(D):
    mesh = plsc.VectorSubcoreMesh(core_axis_name="c", subcore_axis_name="s")

    @functools.partial(
        pl.kernel,
        mesh=mesh,
        out_type=jax.ShapeDtypeStruct((NC, NP, D), jnp.float32),
        scratch_types=[
            pltpu.VMEM((S, B), jnp.int32),
            pltpu.VMEM((S, B), jnp.int32),
            [pltpu.VMEM((B, D), jnp.float32) for _ in range(NBUF)],
            pltpu.VMEM_SHARED((NP, D), jnp.float32),
            [pltpu.SemaphoreType.DMA for _ in range(NBUF)],
        ],
        compiler_params=pltpu.CompilerParams(use_tc_tiling_on_sc=False),
    )
    def seg(f_hbm, src_hbm, dst_hbm, zeros_hbm, out_hbm,
            src_v, dst_v, rows_v, acc_sh, sems):
        c = lax.axis_index("c")
        s = lax.axis_index("s")
        w = c * NS + s

        pltpu.sync_copy(src_hbm.at[w], src_v)
        pltpu.sync_copy(dst_hbm.at[w], dst_v)

        pltpu.sync_copy(zeros_hbm.at[pl.ds(s * RPT, RPT)],
                        acc_sh.at[pl.ds(s * RPT, RPT)])
        plsc.subcore_barrier()

        for b in range(NBUF):
            pltpu.async_copy(f_hbm.at[src_v.at[b]], rows_v[b], sems[b])

        def body(i, carry):
            for b in range(NBUF):
                j = NBUF * i + b
                pltpu.make_async_copy(f_hbm.at[src_v.at[j]], rows_v[b],
                                      sems[b]).wait()
                pltpu.sync_copy(rows_v[b], acc_sh.at[dst_v.at[j]],
                                add=True)
                jn = jnp.minimum(j + NBUF, S - 1)

                @pl.when(j + NBUF < S)
                def _():
                    pltpu.async_copy(f_hbm.at[src_v.at[jn]], rows_v[b],
                                     sems[b])
            return carry

        lax.fori_loop(0, S // NBUF, body, 0)
        plsc.subcore_barrier()

        pltpu.sync_copy(acc_sh.at[pl.ds(s * RPT, RPT)],
                        out_hbm.at[c, pl.ds(s * RPT, RPT)])

    return seg


_segsum_l1 = _make_segsum(D1)
_segsum_l2 = _make_segsum(OUT_CH)


_BN1 = 2048


def _tc1_body(p_ref, x_ref, wl1_ref, wr1_ref, b1_ref, wl2_ref, wr2_ref,
              b2_ref, y2_ref, r2_ref, inv_ref):
    acc = p_ref[0] + p_ref[1]
    inv = 1.0 / jnp.maximum(acc[:, IN_CH:IN_CH + 1], 1.0)
    agg = acc[:, :IN_CH] * inv
    h = jnp.maximum(
        jnp.dot(agg, wl1_ref[...], preferred_element_type=jnp.float32)
        + jnp.dot(x_ref[...], wr1_ref[...], preferred_element_type=jnp.float32)
        + b1_ref[...], 0.0)
    y2_ref[...] = jnp.dot(h, wl2_ref[...], preferred_element_type=jnp.float32)
    r2_ref[...] = (jnp.dot(h, wr2_ref[...], preferred_element_type=jnp.float32)
                   + b2_ref[...])
    inv_ref[...] = inv


def _tc1(p1, x, wl1, wr1, b1, wl2, wr2, b2):
    g = NP // _BN1
    return pl.pallas_call(
        _tc1_body,
        grid=(g,),
        in_specs=[
            pl.BlockSpec((NC, _BN1, D1), lambda i: (0, i, 0)),
            pl.BlockSpec((_BN1, IN_CH), lambda i: (i, 0)),
            pl.BlockSpec((IN_CH, HIDDEN), lambda i: (0, 0)),
            pl.BlockSpec((IN_CH, HIDDEN), lambda i: (0, 0)),
            pl.BlockSpec((1, HIDDEN), lambda i: (0, 0)),
            pl.BlockSpec((HIDDEN, OUT_CH), lambda i: (0, 0)),
            pl.BlockSpec((HIDDEN, OUT_CH), lambda i: (0, 0)),
            pl.BlockSpec((1, OUT_CH), lambda i: (0, 0)),
        ],
        out_specs=[
            pl.BlockSpec((_BN1, OUT_CH), lambda i: (i, 0)),
            pl.BlockSpec((_BN1, OUT_CH), lambda i: (i, 0)),
            pl.BlockSpec((_BN1, 1), lambda i: (i, 0)),
        ],
        out_shape=[
            jax.ShapeDtypeStruct((NP, OUT_CH), jnp.float32),
            jax.ShapeDtypeStruct((NP, OUT_CH), jnp.float32),
            jax.ShapeDtypeStruct((NP, 1), jnp.float32),
        ],
    )(p1, x, wl1, wr1, b1, wl2, wr2, b2)


_BN2 = 2000


def _tc2_body(p_ref, r2_ref, inv_ref, z_ref):
    z_ref[...] = (p_ref[0] + p_ref[1]) * inv_ref[...] + r2_ref[...]


def _tc2(p2, r2, inv):
    g = N // _BN2
    return pl.pallas_call(
        _tc2_body,
        grid=(g,),
        in_specs=[
            pl.BlockSpec((NC, _BN2, OUT_CH), lambda i: (0, i, 0)),
            pl.BlockSpec((_BN2, OUT_CH), lambda i: (i, 0)),
            pl.BlockSpec((_BN2, 1), lambda i: (i, 0)),
        ],
        out_specs=pl.BlockSpec((_BN2, OUT_CH), lambda i: (i, 0)),
        out_shape=jax.ShapeDtypeStruct((N, OUT_CH), jnp.float32),
    )(p2, r2, inv)


def kernel(x, edge_index, Wl1, Wr1, b1, Wl2, Wr2, b2):
    npad = EPAD - E
    fill = (N + (jnp.arange(npad, dtype=jnp.int32) % (NP - N)))
    src = jnp.concatenate([edge_index[0], fill]).reshape(NW, S, B)
    dst = jnp.concatenate([edge_index[1], fill]).reshape(NW, S, B)

    x_pad = jnp.pad(x, ((0, NP - N), (0, 0)))
    ones_col = jnp.pad(jnp.ones((N, 1), jnp.float32), ((0, NP - N), (0, 0)))
    x_aug = jnp.concatenate(
        [x_pad, ones_col, jnp.zeros((NP, D1 - IN_CH - 1), jnp.float32)],
        axis=1)
    zeros1 = jnp.zeros((NP, D1), jnp.float32)
    p1 = _segsum_l1(x_aug, src, dst, zeros1)

    y2, r2, inv = _tc1(p1, x_pad, Wl1, Wr1, b1.reshape(1, HIDDEN),
                       Wl2, Wr2, b2.reshape(1, OUT_CH))

    zeros2 = jnp.zeros((NP, OUT_CH), jnp.float32)
    p2 = _segsum_l2(y2, src, dst, zeros2)

    return _tc2(p2, r2, inv)
